# Initial kernel scaffold; baseline (speedup 1.0000x reference)
import jax
import jax.numpy as jnp
from jax.experimental import pallas as pl


def kernel(h, edge_index, edge_attr, W1, b1, W2, b2, Wa, ba, Wu1, bu1, Wu2, bu2):
    raise NotImplementedError("write your pallas kernel here")


# speedup vs baseline: 2.5800x; 2.5800x over previous
import jax
import jax.numpy as jnp
from jax import lax
from jax.experimental import pallas as pl
from jax.experimental.pallas import tpu as pltpu
from jax.experimental.pallas import tpu_sc as plsc

N_NODES = 10000
N_EDGES = 320000
D = 128
DE = 16
H = 128
NORM = 32.0

NC = 2
NS = 16
NW = NC * NS
EPW = N_EDGES // NW
CH = 80
NCH = EPW // CH
EPT = N_EDGES // NS
NCH_S = EPT // CH
N_PAD = 10240
NPT = N_PAD // NS
LANES = 16

_mesh2 = plsc.VectorSubcoreMesh(
    core_axis_name="c", subcore_axis_name="s", num_cores=NC, num_subcores=NS)
_mesh1 = plsc.VectorSubcoreMesh(
    core_axis_name="c", subcore_axis_name="s", num_cores=1, num_subcores=NS)


def _gather_body(a_hbm, b_hbm, rowi_hbm, coli_hbm, out_hbm,
                 idxr_v, idxc_v, bufa_v, bufb_v, sem):
    cid = lax.axis_index("c")
    sid = lax.axis_index("s")
    wid = sid * NC + cid
    base = wid * EPW
    pltpu.sync_copy(rowi_hbm.at[wid], idxr_v)
    pltpu.sync_copy(coli_hbm.at[wid], idxc_v)

    def chunk(j, carry):
        pltpu.async_copy(a_hbm.at[idxr_v.at[j]], bufa_v, sem).wait()
        pltpu.async_copy(b_hbm.at[idxc_v.at[j]], bufb_v, sem).wait()

        def addrow(r, c2):
            for c8 in range(D // LANES):
                s = pl.ds(c8 * LANES, LANES)
                bufa_v[r, s] = bufa_v[r, s] + bufb_v[r, s]
            return c2

        lax.fori_loop(0, CH, addrow, 0)
        pltpu.sync_copy(bufa_v, out_hbm.at[pl.ds(base + j * CH, CH)])
        return carry

    lax.fori_loop(0, NCH, chunk, 0)


_gather = pl.kernel(
    _gather_body,
    out_type=jax.ShapeDtypeStruct((N_EDGES, D), jnp.float32),
    mesh=_mesh2,
    scratch_types=[
        pltpu.VMEM((NCH, CH), jnp.int32),
        pltpu.VMEM((NCH, CH), jnp.int32),
        pltpu.VMEM((CH, D), jnp.float32),
        pltpu.VMEM((CH, D), jnp.float32),
        pltpu.SemaphoreType.DMA,
    ],
)


def _scatter_body(m_hbm, rowi_hbm, out_hbm, idxr_v, mbuf_v, acc_s, sem):
    sid = lax.axis_index("s")
    base = sid * EPT

    def zrow(r, carry):
        for c8 in range(D // LANES):
            mbuf_v[r, pl.ds(c8 * LANES, LANES)] = jnp.zeros((LANES,), jnp.float32)
        return carry

    lax.fori_loop(0, CH, zrow, 0)

    def zchunk(k, carry):
        pltpu.sync_copy(mbuf_v, acc_s.at[pl.ds(sid * NPT + k * CH, CH)])
        return carry

    lax.fori_loop(0, NPT // CH, zchunk, 0)
    plsc.subcore_barrier()

    pltpu.sync_copy(rowi_hbm.at[sid], idxr_v)

    def chunk(j, carry):
        pltpu.sync_copy(m_hbm.at[pl.ds(base + j * CH, CH)], mbuf_v)
        pltpu.sync_copy(mbuf_v, acc_s.at[idxr_v.at[j]], add=True)
        return carry

    lax.fori_loop(0, NCH_S, chunk, 0)
    plsc.subcore_barrier()

    def dchunk(k, carry):
        pltpu.sync_copy(acc_s.at[pl.ds(sid * NPT + k * CH, CH)], mbuf_v)
        pltpu.sync_copy(mbuf_v, out_hbm.at[pl.ds(sid * NPT + k * CH, CH)])
        return carry

    lax.fori_loop(0, NPT // CH, dchunk, 0)


_scatter = pl.kernel(
    _scatter_body,
    out_type=jax.ShapeDtypeStruct((N_PAD, D), jnp.float32),
    mesh=_mesh1,
    scratch_types=[
        pltpu.VMEM((NCH_S, CH), jnp.int32),
        pltpu.VMEM((CH, D), jnp.float32),
        pltpu.MemorySpace.VMEM_SHARED((N_PAD, D), jnp.float32),
        pltpu.SemaphoreType.DMA,
    ],
)


def _precompute_body(h_ref, w1a_ref, w1b_ref, b1_ref, a_ref, b_ref):
    hh = h_ref[...]
    a_ref[...] = jnp.dot(hh, w1a_ref[...],
                         preferred_element_type=jnp.float32) + b1_ref[...]
    b_ref[...] = jnp.dot(hh, w1b_ref[...], preferred_element_type=jnp.float32)


def _edge_mlp_body(pre_ref, ea_ref, w1c_ref, w2_ref, b2_ref, wat_ref, ba_ref,
                   out_ref):
    x = pre_ref[...] + jnp.dot(ea_ref[...], w1c_ref[...],
                               preferred_element_type=jnp.float32)
    m = jnp.maximum(x, 0.0)
    m = jnp.maximum(
        jnp.dot(m, w2_ref[...], preferred_element_type=jnp.float32)
        + b2_ref[...], 0.0)
    logit = jnp.sum(m * wat_ref[...], axis=1, keepdims=True) + ba_ref[...]
    out_ref[...] = m * jax.nn.sigmoid(logit)


def _node_body(h_ref, p_ref, wu1h_ref, wu1a_ref, bu1_ref, wu2_ref, bu2_ref,
               out_ref):
    hh = h_ref[...]
    u = jnp.maximum(
        jnp.dot(hh, wu1h_ref[...], preferred_element_type=jnp.float32)
        + jnp.dot(p_ref[...] * (1.0 / NORM), wu1a_ref[...],
                  preferred_element_type=jnp.float32)
        + bu1_ref[...], 0.0)
    out_ref[...] = hh + jnp.dot(u, wu2_ref[...],
                                preferred_element_type=jnp.float32) + bu2_ref[...]


BE = 8000
BN = 1000


def kernel(h, edge_index, edge_attr, W1, b1, W2, b2, Wa, ba, Wu1, bu1, Wu2, bu2):
    W1a, W1b, W1c = W1[:D], W1[D:2 * D], W1[2 * D:]
    Wu1h, Wu1a = Wu1[:D], Wu1[D:]
    row = edge_index[0]
    row3 = row.reshape(NW, NCH, CH)
    col3 = edge_index[1].reshape(NW, NCH, CH)
    row3s = row.reshape(NS, NCH_S, CH)

    A, B = pl.pallas_call(
        _precompute_body,
        out_shape=[jax.ShapeDtypeStruct((N_NODES, D), jnp.float32),
                   jax.ShapeDtypeStruct((N_NODES, D), jnp.float32)],
    )(h, W1a, W1b, b1.reshape(1, H))

    pre = _gather(A, B, row3, col3)

    zero = lambda i: (0, 0)
    mm = pl.pallas_call(
        _edge_mlp_body,
        grid=(N_EDGES // BE,),
        in_specs=[
            pl.BlockSpec((BE, D), lambda i: (i, 0)),
            pl.BlockSpec((BE, DE), lambda i: (i, 0)),
            pl.BlockSpec((DE, H), zero),
            pl.BlockSpec((H, H), zero),
            pl.BlockSpec((1, H), zero),
            pl.BlockSpec((1, H), zero),
            pl.BlockSpec((1, 1), zero),
        ],
        out_specs=pl.BlockSpec((BE, D), lambda i: (i, 0)),
        out_shape=jax.ShapeDtypeStruct((N_EDGES, D), jnp.float32),
    )(pre, edge_attr, W1c, W2, b2.reshape(1, H), Wa.reshape(1, H),
      ba.reshape(1, 1))

    agg = _scatter(mm, row3s)[:N_NODES]

    out = pl.pallas_call(
        _node_body,
        grid=(N_NODES // BN,),
        in_specs=[
            pl.BlockSpec((BN, D), lambda i: (i, 0)),
            pl.BlockSpec((BN, D), lambda i: (i, 0)),
            pl.BlockSpec((H, H), zero),
            pl.BlockSpec((H, H), zero),
            pl.BlockSpec((1, H), zero),
            pl.BlockSpec((H, H), zero),
            pl.BlockSpec((1, H), zero),
        ],
        out_specs=pl.BlockSpec((BN, D), lambda i: (i, 0)),
        out_shape=jax.ShapeDtypeStruct((N_NODES, D), jnp.float32),
    )(h, agg, Wu1h, Wu1a, bu1.reshape(1, H), Wu2, bu2.reshape(1, H))

    return out


# speedup vs baseline: 2.7669x; 1.0724x over previous
import jax
import jax.numpy as jnp
from jax import lax
from jax.experimental import pallas as pl
from jax.experimental.pallas import tpu as pltpu
from jax.experimental.pallas import tpu_sc as plsc

N_NODES = 10000
N_EDGES = 320000
D = 128
DE = 16
H = 128
NORM = 32.0

NC = 2
NS = 16
NW = NC * NS
EPW = N_EDGES // NW
CH = 80
NCH = EPW // CH
EPT = N_EDGES // NS
NCH_S = EPT // CH
N_PAD = 10240
NPT = N_PAD // NS
LANES = 16

_mesh2 = plsc.VectorSubcoreMesh(
    core_axis_name="c", subcore_axis_name="s", num_cores=NC, num_subcores=NS)
_mesh1 = plsc.VectorSubcoreMesh(
    core_axis_name="c", subcore_axis_name="s", num_cores=1, num_subcores=NS)


def _gather_body(a_hbm, b_hbm, rowi_hbm, coli_hbm, out_hbm,
                 idxr_v, idxc_v, bufa_v, sem):
    cid = lax.axis_index("c")
    sid = lax.axis_index("s")
    wid = sid * NC + cid
    base = wid * EPW
    pltpu.sync_copy(rowi_hbm.at[wid], idxr_v)
    pltpu.sync_copy(coli_hbm.at[wid], idxc_v)

    def chunk(j, carry):
        pltpu.async_copy(a_hbm.at[idxr_v.at[j]], bufa_v, sem).wait()
        pltpu.async_copy(b_hbm.at[idxc_v.at[j]], bufa_v, sem, add=True).wait()
        pltpu.sync_copy(bufa_v, out_hbm.at[pl.ds(base + j * CH, CH)])
        return carry

    lax.fori_loop(0, NCH, chunk, 0)


_gather = pl.kernel(
    _gather_body,
    out_type=jax.ShapeDtypeStruct((N_EDGES, D), jnp.float32),
    mesh=_mesh2,
    scratch_types=[
        pltpu.VMEM((NCH, CH), jnp.int32),
        pltpu.VMEM((NCH, CH), jnp.int32),
        pltpu.VMEM((CH, D), jnp.float32),
        pltpu.SemaphoreType.DMA,
    ],
)


def _scatter_body(m_hbm, rowi_hbm, out_hbm, idxr_v, mbuf_v, acc_s, sem):
    sid = lax.axis_index("s")
    base = sid * EPT

    def zrow(r, carry):
        for c8 in range(D // LANES):
            mbuf_v[r, pl.ds(c8 * LANES, LANES)] = jnp.zeros((LANES,), jnp.float32)
        return carry

    lax.fori_loop(0, CH, zrow, 0)

    def zchunk(k, carry):
        pltpu.sync_copy(mbuf_v, acc_s.at[pl.ds(sid * NPT + k * CH, CH)])
        return carry

    lax.fori_loop(0, NPT // CH, zchunk, 0)
    plsc.subcore_barrier()

    pltpu.sync_copy(rowi_hbm.at[sid], idxr_v)

    def chunk(j, carry):
        pltpu.sync_copy(m_hbm.at[pl.ds(base + j * CH, CH)], mbuf_v)
        pltpu.sync_copy(mbuf_v, acc_s.at[idxr_v.at[j]], add=True)
        return carry

    lax.fori_loop(0, NCH_S, chunk, 0)
    plsc.subcore_barrier()

    def dchunk(k, carry):
        pltpu.sync_copy(acc_s.at[pl.ds(sid * NPT + k * CH, CH)], mbuf_v)
        pltpu.sync_copy(mbuf_v, out_hbm.at[pl.ds(sid * NPT + k * CH, CH)])
        return carry

    lax.fori_loop(0, NPT // CH, dchunk, 0)


_scatter = pl.kernel(
    _scatter_body,
    out_type=jax.ShapeDtypeStruct((N_PAD, D), jnp.float32),
    mesh=_mesh1,
    scratch_types=[
        pltpu.VMEM((NCH_S, CH), jnp.int32),
        pltpu.VMEM((CH, D), jnp.float32),
        pltpu.MemorySpace.VMEM_SHARED((N_PAD, D), jnp.float32),
        pltpu.SemaphoreType.DMA,
    ],
)


def _precompute_body(h_ref, w1a_ref, w1b_ref, b1_ref, a_ref, b_ref):
    hh = h_ref[...]
    a_ref[...] = jnp.dot(hh, w1a_ref[...],
                         preferred_element_type=jnp.float32) + b1_ref[...]
    b_ref[...] = jnp.dot(hh, w1b_ref[...], preferred_element_type=jnp.float32)


def _edge_mlp_body(pre_ref, ea_ref, w1c_ref, w2_ref, b2_ref, wat_ref, ba_ref,
                   out_ref):
    x = pre_ref[...] + jnp.dot(ea_ref[...], w1c_ref[...],
                               preferred_element_type=jnp.float32)
    m = jnp.maximum(x, 0.0)
    m = jnp.maximum(
        jnp.dot(m, w2_ref[...], preferred_element_type=jnp.float32)
        + b2_ref[...], 0.0)
    logit = jnp.sum(m * wat_ref[...], axis=1, keepdims=True) + ba_ref[...]
    out_ref[...] = m * jax.nn.sigmoid(logit)


def _node_body(h_ref, p_ref, wu1h_ref, wu1a_ref, bu1_ref, wu2_ref, bu2_ref,
               out_ref):
    hh = h_ref[...]
    u = jnp.maximum(
        jnp.dot(hh, wu1h_ref[...], preferred_element_type=jnp.float32)
        + jnp.dot(p_ref[...] * (1.0 / NORM), wu1a_ref[...],
                  preferred_element_type=jnp.float32)
        + bu1_ref[...], 0.0)
    out_ref[...] = hh + jnp.dot(u, wu2_ref[...],
                                preferred_element_type=jnp.float32) + bu2_ref[...]


BE = 8000
BN = 1000


def kernel(h, edge_index, edge_attr, W1, b1, W2, b2, Wa, ba, Wu1, bu1, Wu2, bu2):
    W1a, W1b, W1c = W1[:D], W1[D:2 * D], W1[2 * D:]
    Wu1h, Wu1a = Wu1[:D], Wu1[D:]
    row = edge_index[0]
    row3 = row.reshape(NW, NCH, CH)
    col3 = edge_index[1].reshape(NW, NCH, CH)
    row3s = row.reshape(NS, NCH_S, CH)

    A, B = pl.pallas_call(
        _precompute_body,
        out_shape=[jax.ShapeDtypeStruct((N_NODES, D), jnp.float32),
                   jax.ShapeDtypeStruct((N_NODES, D), jnp.float32)],
    )(h, W1a, W1b, b1.reshape(1, H))

    pre = _gather(A, B, row3, col3)

    zero = lambda i: (0, 0)
    mm = pl.pallas_call(
        _edge_mlp_body,
        grid=(N_EDGES // BE,),
        in_specs=[
            pl.BlockSpec((BE, D), lambda i: (i, 0)),
            pl.BlockSpec((BE, DE), lambda i: (i, 0)),
            pl.BlockSpec((DE, H), zero),
            pl.BlockSpec((H, H), zero),
            pl.BlockSpec((1, H), zero),
            pl.BlockSpec((1, H), zero),
            pl.BlockSpec((1, 1), zero),
        ],
        out_specs=pl.BlockSpec((BE, D), lambda i: (i, 0)),
        out_shape=jax.ShapeDtypeStruct((N_EDGES, D), jnp.float32),
    )(pre, edge_attr, W1c, W2, b2.reshape(1, H), Wa.reshape(1, H),
      ba.reshape(1, 1))

    agg = _scatter(mm, row3s)[:N_NODES]

    out = pl.pallas_call(
        _node_body,
        grid=(N_NODES // BN,),
        in_specs=[
            pl.BlockSpec((BN, D), lambda i: (i, 0)),
            pl.BlockSpec((BN, D), lambda i: (i, 0)),
            pl.BlockSpec((H, H), zero),
            pl.BlockSpec((H, H), zero),
            pl.BlockSpec((1, H), zero),
            pl.BlockSpec((H, H), zero),
            pl.BlockSpec((1, H), zero),
        ],
        out_specs=pl.BlockSpec((BN, D), lambda i: (i, 0)),
        out_shape=jax.ShapeDtypeStruct((N_NODES, D), jnp.float32),
    )(h, agg, Wu1h, Wu1a, bu1.reshape(1, H), Wu2, bu2.reshape(1, H))

    return out


# speedup vs baseline: 3.6501x; 1.3192x over previous
import jax
import jax.numpy as jnp
from jax import lax
from jax.experimental import pallas as pl
from jax.experimental.pallas import tpu as pltpu
from jax.experimental.pallas import tpu_sc as plsc

N_NODES = 10000
N_EDGES = 320000
D = 128
DE = 16
H = 128
NORM = 32.0

NC = 2
NS = 16
NW = NC * NS
EPW = N_EDGES // NW
CH = 80
NCH = EPW // CH
EPT = N_EDGES // NS
NCH_S = EPT // CH
KS = 4
N_PAD = 10240
NPT = N_PAD // NS
LANES = 16

_mesh2 = plsc.VectorSubcoreMesh(
    core_axis_name="c", subcore_axis_name="s", num_cores=NC, num_subcores=NS)
_mesh1 = plsc.VectorSubcoreMesh(
    core_axis_name="c", subcore_axis_name="s", num_cores=1, num_subcores=NS)


K = 5
NG = NCH // K


def _gather_body(a_hbm, b_hbm, rowi_hbm, coli_hbm, out_hbm,
                 idxr_v, idxc_v, b0, b1, b2, b3, b4, s0, s1, s2, s3, s4):
    bufs = (b0, b1, b2, b3, b4)
    sems = (s0, s1, s2, s3, s4)
    cid = lax.axis_index("c")
    sid = lax.axis_index("s")
    wid = sid * NC + cid
    base = wid * EPW
    pltpu.sync_copy(rowi_hbm.at[wid], idxr_v)
    pltpu.sync_copy(coli_hbm.at[wid], idxc_v)

    def group(g, carry):
        js = [g * K + b for b in range(K)]
        for b in range(K):
            @pl.when(g > 0)
            def _(b=b):
                pltpu.make_async_copy(
                    bufs[b], out_hbm.at[pl.ds(base, CH)], sems[b]).wait()
            pltpu.async_copy(a_hbm.at[idxr_v.at[js[b]]], bufs[b], sems[b])
        for b in range(K):
            pltpu.make_async_copy(
                a_hbm.at[idxr_v.at[js[b]]], bufs[b], sems[b]).wait()
            pltpu.async_copy(b_hbm.at[idxc_v.at[js[b]]], bufs[b], sems[b],
                             add=True)
        for b in range(K):
            pltpu.make_async_copy(
                b_hbm.at[idxc_v.at[js[b]]], bufs[b], sems[b]).wait()
            pltpu.async_copy(bufs[b],
                             out_hbm.at[pl.ds(base + js[b] * CH, CH)], sems[b])
        return carry

    lax.fori_loop(0, NG, group, 0)
    for b in range(K):
        pltpu.make_async_copy(bufs[b], out_hbm.at[pl.ds(base, CH)],
                              sems[b]).wait()


_gather = pl.kernel(
    _gather_body,
    out_type=jax.ShapeDtypeStruct((N_EDGES, D), jnp.float32),
    mesh=_mesh2,
    scratch_types=[
        pltpu.VMEM((NCH, CH), jnp.int32),
        pltpu.VMEM((NCH, CH), jnp.int32),
    ] + [pltpu.VMEM((CH, D), jnp.float32)] * K
      + [pltpu.SemaphoreType.DMA] * K,
)


def _scatter_body(m_hbm, rowi_hbm, out_hbm, idxr_v, mbuf_v, mbuf2_v, acc_s, sem):
    sid = lax.axis_index("s")
    base = sid * EPT

    def zrow(r, carry):
        for c8 in range(D // LANES):
            mbuf_v[r, pl.ds(c8 * LANES, LANES)] = jnp.zeros((LANES,), jnp.float32)
        return carry

    lax.fori_loop(0, CH, zrow, 0)

    def zchunk(k, carry):
        pltpu.sync_copy(mbuf_v, acc_s.at[pl.ds(sid * NPT + k * CH, CH)])
        return carry

    lax.fori_loop(0, NPT // CH, zchunk, 0)
    plsc.subcore_barrier()

    pltpu.sync_copy(rowi_hbm.at[sid, 0], idxr_v)
    pltpu.async_copy(m_hbm.at[pl.ds(base, CH)], mbuf_v, sem)

    def chunk(j, carry):
        jh = lax.rem(j, NCH_S // 2)

        @pl.when(jnp.logical_and(jh == 0, j > 0))
        def _():
            pltpu.sync_copy(rowi_hbm.at[sid, j // (NCH_S // 2)], idxr_v)

        nxt = jnp.minimum(j + 1, NCH_S - 1)
        cur_is_0 = lax.rem(j, 2) == 0

        @pl.when(cur_is_0)
        def _():
            pltpu.make_async_copy(m_hbm.at[pl.ds(base, CH)], mbuf_v, sem).wait()
            pltpu.async_copy(m_hbm.at[pl.ds(base + nxt * CH, CH)], mbuf2_v, sem)
            pltpu.sync_copy(mbuf_v, acc_s.at[idxr_v.at[jh]], add=True)

        @pl.when(jnp.logical_not(cur_is_0))
        def _():
            pltpu.make_async_copy(m_hbm.at[pl.ds(base, CH)], mbuf2_v, sem).wait()
            pltpu.async_copy(m_hbm.at[pl.ds(base + nxt * CH, CH)], mbuf_v, sem)
            pltpu.sync_copy(mbuf2_v, acc_s.at[idxr_v.at[jh]], add=True)

        return carry

    lax.fori_loop(0, NCH_S, chunk, 0)
    pltpu.make_async_copy(
        m_hbm.at[pl.ds(base, CH)],
        mbuf_v if NCH_S % 2 == 0 else mbuf2_v, sem).wait()
    plsc.subcore_barrier()

    def dchunk(k, carry):
        pltpu.sync_copy(acc_s.at[pl.ds(sid * NPT + k * CH, CH)], mbuf_v)
        pltpu.sync_copy(mbuf_v, out_hbm.at[pl.ds(sid * NPT + k * CH, CH)])
        return carry

    lax.fori_loop(0, NPT // CH, dchunk, 0)


_scatter = pl.kernel(
    _scatter_body,
    out_type=jax.ShapeDtypeStruct((N_PAD, D), jnp.float32),
    mesh=_mesh1,
    scratch_types=[
        pltpu.VMEM((NCH_S // 2, CH), jnp.int32),
        pltpu.VMEM((CH, D), jnp.float32),
        pltpu.VMEM((CH, D), jnp.float32),
        pltpu.MemorySpace.VMEM_SHARED((N_PAD, D), jnp.float32),
        pltpu.SemaphoreType.DMA,
    ],
)


def _precompute_body(h_ref, w1a_ref, w1b_ref, b1_ref, a_ref, b_ref):
    hh = h_ref[...]
    a_ref[...] = jnp.dot(hh, w1a_ref[...],
                         preferred_element_type=jnp.float32) + b1_ref[...]
    b_ref[...] = jnp.dot(hh, w1b_ref[...], preferred_element_type=jnp.float32)


def _edge_mlp_body(pre_ref, ea_ref, w1c_ref, w2_ref, b2_ref, wat_ref, ba_ref,
                   out_ref):
    x = pre_ref[...] + jnp.dot(ea_ref[...], w1c_ref[...],
                               preferred_element_type=jnp.float32)
    m = jnp.maximum(x, 0.0)
    m = jnp.maximum(
        jnp.dot(m, w2_ref[...], preferred_element_type=jnp.float32)
        + b2_ref[...], 0.0)
    logit = jnp.sum(m * wat_ref[...], axis=1, keepdims=True) + ba_ref[...]
    out_ref[...] = m * jax.nn.sigmoid(logit)


def _node_body(h_ref, p_ref, wu1h_ref, wu1a_ref, bu1_ref, wu2_ref, bu2_ref,
               out_ref):
    hh = h_ref[...]
    u = jnp.maximum(
        jnp.dot(hh, wu1h_ref[...], preferred_element_type=jnp.float32)
        + jnp.dot(p_ref[...] * (1.0 / NORM), wu1a_ref[...],
                  preferred_element_type=jnp.float32)
        + bu1_ref[...], 0.0)
    out_ref[...] = hh + jnp.dot(u, wu2_ref[...],
                                preferred_element_type=jnp.float32) + bu2_ref[...]


BE = 8000
BN = 1000


def kernel(h, edge_index, edge_attr, W1, b1, W2, b2, Wa, ba, Wu1, bu1, Wu2, bu2):
    W1a, W1b, W1c = W1[:D], W1[D:2 * D], W1[2 * D:]
    Wu1h, Wu1a = Wu1[:D], Wu1[D:]
    row = edge_index[0]
    row3 = row.reshape(NW, NCH, CH)
    col3 = edge_index[1].reshape(NW, NCH, CH)
    row3s = row.reshape(NS, 2, NCH_S // 2, CH)

    A, B = pl.pallas_call(
        _precompute_body,
        out_shape=[jax.ShapeDtypeStruct((N_NODES, D), jnp.float32),
                   jax.ShapeDtypeStruct((N_NODES, D), jnp.float32)],
    )(h, W1a, W1b, b1.reshape(1, H))

    pre = _gather(A, B, row3, col3)

    zero = lambda i: (0, 0)
    mm = pl.pallas_call(
        _edge_mlp_body,
        grid=(N_EDGES // BE,),
        in_specs=[
            pl.BlockSpec((BE, D), lambda i: (i, 0)),
            pl.BlockSpec((BE, DE), lambda i: (i, 0)),
            pl.BlockSpec((DE, H), zero),
            pl.BlockSpec((H, H), zero),
            pl.BlockSpec((1, H), zero),
            pl.BlockSpec((1, H), zero),
            pl.BlockSpec((1, 1), zero),
        ],
        out_specs=pl.BlockSpec((BE, D), lambda i: (i, 0)),
        out_shape=jax.ShapeDtypeStruct((N_EDGES, D), jnp.float32),
    )(pre, edge_attr, W1c, W2, b2.reshape(1, H), Wa.reshape(1, H),
      ba.reshape(1, 1))

    agg = _scatter(mm, row3s)[:N_NODES]

    out = pl.pallas_call(
        _node_body,
        grid=(N_NODES // BN,),
        in_specs=[
            pl.BlockSpec((BN, D), lambda i: (i, 0)),
            pl.BlockSpec((BN, D), lambda i: (i, 0)),
            pl.BlockSpec((H, H), zero),
            pl.BlockSpec((H, H), zero),
            pl.BlockSpec((1, H), zero),
            pl.BlockSpec((H, H), zero),
            pl.BlockSpec((1, H), zero),
        ],
        out_specs=pl.BlockSpec((BN, D), lambda i: (i, 0)),
        out_shape=jax.ShapeDtypeStruct((N_NODES, D), jnp.float32),
    )(h, agg, Wu1h, Wu1a, bu1.reshape(1, H), Wu2, bu2.reshape(1, H))

    return out
